# baseline (speedup 1.0000x reference)
import functools

import jax
import jax.numpy as jnp
from jax import lax
from jax.experimental import pallas as pl
from jax.experimental.pallas import tpu as pltpu
from jax.experimental.pallas import tpu_sc as plsc

N = 10000
E = 320000
D = 128

NC = 2
NS = 16
NW = NC * NS
CHUNK = 128
NBW = 2 * ((E + NW * 2 * CHUNK - 1) // (NW * 2 * CHUNK))
EPW = NBW * CHUNK
E_PAD = EPW * NW
E_ALL = E_PAD
NBW0 = 62
NBW1 = 2 * NBW - NBW0
ACC_ROWS = 10240
ZROWS = 64
ROWS_PER_TILE = ACC_ROWS // NS


def _spmm_sc(h, src_p, dst_p):
  mesh = plsc.VectorSubcoreMesh(core_axis_name="c", subcore_axis_name="s")

  @functools.partial(
      pl.kernel,
      out_type=jax.ShapeDtypeStruct((NC, ACC_ROWS, D), jnp.float32),
      mesh=mesh,
      scratch_types=[
          pltpu.VMEM((CHUNK,), jnp.int32),
          pltpu.VMEM((CHUNK,), jnp.int32),
          pltpu.VMEM((CHUNK,), jnp.int32),
          pltpu.VMEM((CHUNK,), jnp.int32),
          pltpu.VMEM((CHUNK, D), jnp.float32),
          pltpu.VMEM((CHUNK, D), jnp.float32),
          pltpu.VMEM((ZROWS, D), jnp.float32),
          pltpu.VMEM_SHARED((ACC_ROWS, D), jnp.float32),
          pltpu.SemaphoreType.DMA,
          pltpu.SemaphoreType.DMA,
      ],
  )
  def k(h_hbm, src_hbm, dst_hbm, out_hbm, srcg0, srcg1, dstg0, dstg1,
        rows0, rows1, zbuf, acc, sem0, sem1):
    cid = lax.axis_index("c")
    sid = lax.axis_index("s")

    zero = jnp.zeros((16,), jnp.float32)

    def zrow(i, _):
      zbuf[i // (D // 16), pl.ds((i % (D // 16)) * 16, 16)] = zero
      return 0

    lax.fori_loop(0, ZROWS * (D // 16), zrow, 0)

    def zacc(j, _):
      pltpu.sync_copy(zbuf, acc.at[pl.ds(sid * ROWS_PER_TILE + j * ZROWS, ZROWS)])
      return 0

    lax.fori_loop(0, ROWS_PER_TILE // ZROWS, zacc, 0)
    plsc.subcore_barrier()

    base = lax.select(cid == 0, sid * (NBW0 * CHUNK),
                      NS * (NBW0 * CHUNK) + sid * (NBW1 * CHUNK))
    nb = lax.select(cid == 0, NBW0, NBW1)

    def body(i, _):
      off = pl.multiple_of(base + i * CHUNK, CHUNK)
      pltpu.sync_copy(src_hbm.at[pl.ds(off, CHUNK)], srcg0)
      pltpu.sync_copy(dst_hbm.at[pl.ds(off, CHUNK)], dstg0)
      pltpu.async_copy(h_hbm.at[srcg0], rows0, sem0).wait()
      pltpu.sync_copy(rows0, acc.at[dstg0], add=True)
      return 0

    lax.fori_loop(0, nb, body, 0)
    plsc.subcore_barrier()

    pltpu.sync_copy(acc.at[pl.ds(sid * ROWS_PER_TILE, ROWS_PER_TILE)],
                    out_hbm.at[cid, pl.ds(sid * ROWS_PER_TILE, ROWS_PER_TILE)])

  return k(h, src_p, dst_p)


_BLK = 1000


def _lin1_body(x_ref, w_ref, b_ref, o_ref):
  o_ref[...] = lax.dot_general(
      x_ref[...], w_ref[...], (((1,), (1,)), ((), ())),
      preferred_element_type=jnp.float32) + b_ref[...]


def _lin2_body(p0_ref, p1_ref, w_ref, b_ref, o_ref):
  z = jnp.maximum(p0_ref[...] + p1_ref[...], 0.0)
  o_ref[...] = lax.dot_general(
      z, w_ref[...], (((1,), (1,)), ((), ())),
      preferred_element_type=jnp.float32) + b_ref[...]


def _add_body(a_ref, b_ref, o_ref):
  o_ref[...] = a_ref[...] + b_ref[...]


def _linear1(x, W, b):
  return pl.pallas_call(
      _lin1_body,
      grid=(N // _BLK,),
      in_specs=[
          pl.BlockSpec((_BLK, D), lambda i: (i, 0)),
          pl.BlockSpec((D, D), lambda i: (0, 0)),
          pl.BlockSpec((1, D), lambda i: (0, 0)),
      ],
      out_specs=pl.BlockSpec((_BLK, D), lambda i: (i, 0)),
      out_shape=jax.ShapeDtypeStruct((N, D), jnp.float32),
  )(x, W, b)


def _linear2(p0, p1, W, b):
  return pl.pallas_call(
      _lin2_body,
      grid=(N // _BLK,),
      in_specs=[
          pl.BlockSpec((_BLK, D), lambda i: (i, 0)),
          pl.BlockSpec((_BLK, D), lambda i: (i, 0)),
          pl.BlockSpec((D, D), lambda i: (0, 0)),
          pl.BlockSpec((1, D), lambda i: (0, 0)),
      ],
      out_specs=pl.BlockSpec((_BLK, D), lambda i: (i, 0)),
      out_shape=jax.ShapeDtypeStruct((N, D), jnp.float32),
  )(p0, p1, W, b)


def _add(a, b):
  return pl.pallas_call(
      _add_body,
      grid=(N // _BLK,),
      in_specs=[
          pl.BlockSpec((_BLK, D), lambda i: (i, 0)),
          pl.BlockSpec((_BLK, D), lambda i: (i, 0)),
      ],
      out_specs=pl.BlockSpec((_BLK, D), lambda i: (i, 0)),
      out_shape=jax.ShapeDtypeStruct((N, D), jnp.float32),
  )(a, b)


def kernel(x, edge_index, W1, b1, W2, b2):
  dst = edge_index[0]
  src = edge_index[1]
  pad = E_ALL - E
  src_p = jnp.concatenate([src, jnp.zeros((pad,), jnp.int32)])
  dst_p = jnp.concatenate([dst, jnp.full((pad,), N, jnp.int32)])
  b1r = b1.reshape(1, D)
  b2r = b2.reshape(1, D)

  h1 = _linear1(x, W1, b1r)
  P1 = _spmm_sc(h1, src_p, dst_p)
  h2 = _linear2(P1[0], P1[1], W2, b2r)
  P2 = _spmm_sc(h2, src_p, dst_p)
  return _add(P2[0], P2[1])


# speedup vs baseline: 1.1788x; 1.1788x over previous
import functools

import jax
import jax.numpy as jnp
from jax import lax
from jax.experimental import pallas as pl
from jax.experimental.pallas import tpu as pltpu
from jax.experimental.pallas import tpu_sc as plsc

N = 10000
E = 320000
D = 128

NC = 2
NS = 16
NW = NC * NS
CHUNK = 128
NBW = 2 * ((E + NW * 2 * CHUNK - 1) // (NW * 2 * CHUNK))
EPW = NBW * CHUNK
E_PAD = EPW * NW
E_ALL = E_PAD
NBW0 = 98
NBW1 = 2 * NBW - NBW0
ACC_ROWS = 10240
ZROWS = 64
ROWS_PER_TILE = ACC_ROWS // NS


def _spmm_sc(h, src_p, dst_p):
  mesh = plsc.VectorSubcoreMesh(core_axis_name="c", subcore_axis_name="s")

  @functools.partial(
      pl.kernel,
      out_type=jax.ShapeDtypeStruct((NC, ACC_ROWS, D), jnp.float32),
      mesh=mesh,
      scratch_types=[
          pltpu.VMEM((CHUNK,), jnp.int32),
          pltpu.VMEM((CHUNK,), jnp.int32),
          pltpu.VMEM((CHUNK,), jnp.int32),
          pltpu.VMEM((CHUNK,), jnp.int32),
          pltpu.VMEM((CHUNK, D), jnp.float32),
          pltpu.VMEM((CHUNK, D), jnp.float32),
          pltpu.VMEM((ZROWS, D), jnp.float32),
          pltpu.VMEM_SHARED((ACC_ROWS, D), jnp.float32),
          pltpu.SemaphoreType.DMA,
          pltpu.SemaphoreType.DMA,
      ],
  )
  def k(h_hbm, src_hbm, dst_hbm, out_hbm, srcg0, srcg1, dstg0, dstg1,
        rows0, rows1, zbuf, acc, sem0, sem1):
    cid = lax.axis_index("c")
    sid = lax.axis_index("s")

    zero = jnp.zeros((16,), jnp.float32)

    def zrow(i, _):
      zbuf[i // (D // 16), pl.ds((i % (D // 16)) * 16, 16)] = zero
      return 0

    lax.fori_loop(0, ZROWS * (D // 16), zrow, 0)

    def zacc(j, _):
      pltpu.sync_copy(zbuf, acc.at[pl.ds(sid * ROWS_PER_TILE + j * ZROWS, ZROWS)])
      return 0

    lax.fori_loop(0, ROWS_PER_TILE // ZROWS, zacc, 0)
    plsc.subcore_barrier()

    base = lax.select(cid == 0, sid * (NBW0 * CHUNK),
                      NS * (NBW0 * CHUNK) + sid * (NBW1 * CHUNK))
    nb = lax.select(cid == 0, NBW0, NBW1)

    def body(i, _):
      off = pl.multiple_of(base + i * CHUNK, CHUNK)
      pltpu.sync_copy(src_hbm.at[pl.ds(off, CHUNK)], srcg0)
      pltpu.sync_copy(dst_hbm.at[pl.ds(off, CHUNK)], dstg0)
      pltpu.async_copy(h_hbm.at[srcg0], rows0, sem0).wait()
      pltpu.sync_copy(rows0, acc.at[dstg0], add=True)
      return 0

    lax.fori_loop(0, nb, body, 0)
    plsc.subcore_barrier()

    pltpu.sync_copy(acc.at[pl.ds(sid * ROWS_PER_TILE, ROWS_PER_TILE)],
                    out_hbm.at[cid, pl.ds(sid * ROWS_PER_TILE, ROWS_PER_TILE)])

  return k(h, src_p, dst_p)


_BLK = 1000


def _lin1_body(x_ref, w_ref, b_ref, o_ref):
  o_ref[...] = lax.dot_general(
      x_ref[...], w_ref[...], (((1,), (1,)), ((), ())),
      preferred_element_type=jnp.float32) + b_ref[...]


def _lin2_body(p0_ref, p1_ref, w_ref, b_ref, o_ref):
  z = jnp.maximum(p0_ref[...] + p1_ref[...], 0.0)
  o_ref[...] = lax.dot_general(
      z, w_ref[...], (((1,), (1,)), ((), ())),
      preferred_element_type=jnp.float32) + b_ref[...]


def _add_body(a_ref, b_ref, o_ref):
  o_ref[...] = a_ref[...] + b_ref[...]


def _linear1(x, W, b):
  return pl.pallas_call(
      _lin1_body,
      grid=(N // _BLK,),
      in_specs=[
          pl.BlockSpec((_BLK, D), lambda i: (i, 0)),
          pl.BlockSpec((D, D), lambda i: (0, 0)),
          pl.BlockSpec((1, D), lambda i: (0, 0)),
      ],
      out_specs=pl.BlockSpec((_BLK, D), lambda i: (i, 0)),
      out_shape=jax.ShapeDtypeStruct((N, D), jnp.float32),
  )(x, W, b)


def _linear2(p0, p1, W, b):
  return pl.pallas_call(
      _lin2_body,
      grid=(N // _BLK,),
      in_specs=[
          pl.BlockSpec((_BLK, D), lambda i: (i, 0)),
          pl.BlockSpec((_BLK, D), lambda i: (i, 0)),
          pl.BlockSpec((D, D), lambda i: (0, 0)),
          pl.BlockSpec((1, D), lambda i: (0, 0)),
      ],
      out_specs=pl.BlockSpec((_BLK, D), lambda i: (i, 0)),
      out_shape=jax.ShapeDtypeStruct((N, D), jnp.float32),
  )(p0, p1, W, b)


def _add(a, b):
  return pl.pallas_call(
      _add_body,
      grid=(N // _BLK,),
      in_specs=[
          pl.BlockSpec((_BLK, D), lambda i: (i, 0)),
          pl.BlockSpec((_BLK, D), lambda i: (i, 0)),
      ],
      out_specs=pl.BlockSpec((_BLK, D), lambda i: (i, 0)),
      out_shape=jax.ShapeDtypeStruct((N, D), jnp.float32),
  )(a, b)


def kernel(x, edge_index, W1, b1, W2, b2):
  dst = edge_index[0]
  src = edge_index[1]
  pad = E_ALL - E
  src_p = jnp.concatenate([src, jnp.zeros((pad,), jnp.int32)])
  dst_p = jnp.concatenate([dst, jnp.full((pad,), N, jnp.int32)])
  b1r = b1.reshape(1, D)
  b2r = b2.reshape(1, D)

  h1 = _linear1(x, W1, b1r)
  P1 = _spmm_sc(h1, src_p, dst_p)
  h2 = _linear2(P1[0], P1[1], W2, b2r)
  P2 = _spmm_sc(h2, src_p, dst_p)
  return _add(P2[0], P2[1])


# speedup vs baseline: 1.2199x; 1.0348x over previous
import functools

import jax
import jax.numpy as jnp
from jax import lax
from jax.experimental import pallas as pl
from jax.experimental.pallas import tpu as pltpu
from jax.experimental.pallas import tpu_sc as plsc

N = 10000
E = 320000
D = 128

NC = 2
NS = 16
NW = NC * NS
CHUNK = 128
NBW = 8 * ((E + NW * 8 * CHUNK - 1) // (NW * 8 * CHUNK))
EPW = NBW * CHUNK
E_PAD = EPW * NW
ACC_ROWS = 10240
ZROWS = 64
ROWS_PER_TILE = ACC_ROWS // NS


def _spmm_sc(h, src2, dst2):
  mesh = plsc.VectorSubcoreMesh(core_axis_name="c", subcore_axis_name="s")

  @functools.partial(
      pl.kernel,
      out_type=jax.ShapeDtypeStruct((NC, ACC_ROWS, D), jnp.float32),
      mesh=mesh,
      scratch_types=[
          pltpu.VMEM((NBW, CHUNK), jnp.int32),
          pltpu.VMEM((NBW, CHUNK), jnp.int32),
          pltpu.VMEM((CHUNK, D), jnp.float32),
          pltpu.VMEM((ZROWS, D), jnp.float32),
          pltpu.VMEM_SHARED((ACC_ROWS, D), jnp.float32),
          pltpu.SemaphoreType.DMA,
      ],
  )
  def k(h_hbm, src_hbm, dst_hbm, out_hbm, srcv, dstv, rows0, zbuf, acc, sem0):
    cid = lax.axis_index("c")
    sid = lax.axis_index("s")

    zero = jnp.zeros((16,), jnp.float32)

    def zrow(i, _):
      zbuf[i // (D // 16), pl.ds((i % (D // 16)) * 16, 16)] = zero
      return 0

    lax.fori_loop(0, ZROWS * (D // 16), zrow, 0)

    def zacc(j, _):
      pltpu.sync_copy(zbuf, acc.at[pl.ds(sid * ROWS_PER_TILE + j * ZROWS, ZROWS)])
      return 0

    lax.fori_loop(0, ROWS_PER_TILE // ZROWS, zacc, 0)
    plsc.subcore_barrier()

    wid = sid * NC + cid
    bblk = wid * NBW

    pltpu.sync_copy(src_hbm.at[pl.ds(bblk, NBW)], srcv)
    pltpu.sync_copy(dst_hbm.at[pl.ds(bblk, NBW)], dstv)

    def body(i, _):
      pltpu.async_copy(h_hbm.at[srcv.at[i]], rows0, sem0).wait()
      pltpu.sync_copy(rows0, acc.at[dstv.at[i]], add=True)
      return 0

    lax.fori_loop(0, NBW, body, 0)
    plsc.subcore_barrier()

    pltpu.sync_copy(acc.at[pl.ds(sid * ROWS_PER_TILE, ROWS_PER_TILE)],
                    out_hbm.at[cid, pl.ds(sid * ROWS_PER_TILE, ROWS_PER_TILE)])

  return k(h, src2, dst2)


_BLK = 1000


def _lin1_body(x_ref, w_ref, b_ref, o_ref):
  o_ref[...] = lax.dot_general(
      x_ref[...], w_ref[...], (((1,), (1,)), ((), ())),
      preferred_element_type=jnp.float32) + b_ref[...]


def _lin2_body(p0_ref, p1_ref, w_ref, b_ref, o_ref):
  z = jnp.maximum(p0_ref[...] + p1_ref[...], 0.0)
  o_ref[...] = lax.dot_general(
      z, w_ref[...], (((1,), (1,)), ((), ())),
      preferred_element_type=jnp.float32) + b_ref[...]


def _add_body(a_ref, b_ref, o_ref):
  o_ref[...] = a_ref[...] + b_ref[...]


def _linear1(x, W, b):
  return pl.pallas_call(
      _lin1_body,
      grid=(N // _BLK,),
      in_specs=[
          pl.BlockSpec((_BLK, D), lambda i: (i, 0)),
          pl.BlockSpec((D, D), lambda i: (0, 0)),
          pl.BlockSpec((1, D), lambda i: (0, 0)),
      ],
      out_specs=pl.BlockSpec((_BLK, D), lambda i: (i, 0)),
      out_shape=jax.ShapeDtypeStruct((N, D), jnp.float32),
  )(x, W, b)


def _linear2(p0, p1, W, b):
  return pl.pallas_call(
      _lin2_body,
      grid=(N // _BLK,),
      in_specs=[
          pl.BlockSpec((_BLK, D), lambda i: (i, 0)),
          pl.BlockSpec((_BLK, D), lambda i: (i, 0)),
          pl.BlockSpec((D, D), lambda i: (0, 0)),
          pl.BlockSpec((1, D), lambda i: (0, 0)),
      ],
      out_specs=pl.BlockSpec((_BLK, D), lambda i: (i, 0)),
      out_shape=jax.ShapeDtypeStruct((N, D), jnp.float32),
  )(p0, p1, W, b)


def _add(a, b):
  return pl.pallas_call(
      _add_body,
      grid=(N // _BLK,),
      in_specs=[
          pl.BlockSpec((_BLK, D), lambda i: (i, 0)),
          pl.BlockSpec((_BLK, D), lambda i: (i, 0)),
      ],
      out_specs=pl.BlockSpec((_BLK, D), lambda i: (i, 0)),
      out_shape=jax.ShapeDtypeStruct((N, D), jnp.float32),
  )(a, b)


def kernel(x, edge_index, W1, b1, W2, b2):
  dst = edge_index[0]
  src = edge_index[1]
  pad = E_PAD - E
  src_p = jnp.concatenate([src, jnp.zeros((pad,), jnp.int32)])
  dst_p = jnp.concatenate([dst, jnp.full((pad,), N, jnp.int32)])
  src2 = src_p.reshape(-1, CHUNK)
  dst2 = dst_p.reshape(-1, CHUNK)
  b1r = b1.reshape(1, D)
  b2r = b2.reshape(1, D)

  h1 = _linear1(x, W1, b1r)
  P1 = _spmm_sc(h1, src2, dst2)
  h2 = _linear2(P1[0], P1[1], W2, b2r)
  P2 = _spmm_sc(h2, src2, dst2)
  return _add(P2[0], P2[1])


# speedup vs baseline: 1.5768x; 1.2926x over previous
import functools

import jax
import jax.numpy as jnp
from jax import lax
from jax.experimental import pallas as pl
from jax.experimental.pallas import tpu as pltpu
from jax.experimental.pallas import tpu_sc as plsc

N = 10000
E = 320000
D = 128

NC = 2
NS = 16
NW = NC * NS
CHUNK = 128
NBW = (E + NW * CHUNK - 1) // (NW * CHUNK)
EPW = NBW * CHUNK
E_PAD = EPW * NW
ACC_ROWS = 10240
ZROWS = 64
ROWS_PER_TILE = ACC_ROWS // NS


def _spmm_sc(h, src_p, dst_p):
  mesh = plsc.VectorSubcoreMesh(core_axis_name="c", subcore_axis_name="s")

  @functools.partial(
      pl.kernel,
      out_type=jax.ShapeDtypeStruct((NC, ACC_ROWS, D), jnp.float32),
      mesh=mesh,
      scratch_types=[
          pltpu.VMEM((CHUNK,), jnp.int32),
          pltpu.VMEM((CHUNK,), jnp.int32),
          pltpu.VMEM((CHUNK, D), jnp.float32),
          pltpu.VMEM((ZROWS, D), jnp.float32),
          pltpu.VMEM_SHARED((ACC_ROWS, D), jnp.float32),
          pltpu.SemaphoreType.DMA,
      ],
  )
  def k(h_hbm, src_hbm, dst_hbm, out_hbm, sidx, didx, rows, zbuf, acc, sem):
    cid = lax.axis_index("c")
    sid = lax.axis_index("s")

    zero = jnp.zeros((16,), jnp.float32)

    def zrow(i, _):
      zbuf[i // (D // 16), pl.ds((i % (D // 16)) * 16, 16)] = zero
      return 0

    lax.fori_loop(0, ZROWS * (D // 16), zrow, 0)

    def zacc(j, _):
      pltpu.sync_copy(zbuf, acc.at[pl.ds(sid * ROWS_PER_TILE + j * ZROWS, ZROWS)])
      return 0

    lax.fori_loop(0, ROWS_PER_TILE // ZROWS, zacc, 0)
    plsc.subcore_barrier()

    wid = sid * NC + cid
    base = wid * EPW

    def body(i, _):
      off = pl.multiple_of(base + i * CHUNK, CHUNK)
      pltpu.sync_copy(src_hbm.at[pl.ds(off, CHUNK)], sidx)
      pltpu.sync_copy(dst_hbm.at[pl.ds(off, CHUNK)], didx)
      pltpu.async_copy(h_hbm.at[sidx], rows, sem).wait()
      pltpu.sync_copy(rows, acc.at[didx], add=True)
      return 0

    lax.fori_loop(0, NBW, body, 0)
    plsc.subcore_barrier()

    pltpu.sync_copy(acc.at[pl.ds(sid * ROWS_PER_TILE, ROWS_PER_TILE)],
                    out_hbm.at[cid, pl.ds(sid * ROWS_PER_TILE, ROWS_PER_TILE)])

  return k(h, src_p, dst_p)


_BLK = 1000


def _lin1_body(x_ref, w_ref, b_ref, o_ref):
  o_ref[...] = lax.dot_general(
      x_ref[...], w_ref[...], (((1,), (1,)), ((), ())),
      preferred_element_type=jnp.float32) + b_ref[...]


def _lin2_body(p0_ref, p1_ref, w_ref, b_ref, o_ref):
  z = jnp.maximum(p0_ref[...] + p1_ref[...], 0.0)
  o_ref[...] = lax.dot_general(
      z, w_ref[...], (((1,), (1,)), ((), ())),
      preferred_element_type=jnp.float32) + b_ref[...]


def _add_body(a_ref, b_ref, o_ref):
  o_ref[...] = a_ref[...] + b_ref[...]


def _linear1(x, W, b):
  return pl.pallas_call(
      _lin1_body,
      grid=(N // _BLK,),
      in_specs=[
          pl.BlockSpec((_BLK, D), lambda i: (i, 0)),
          pl.BlockSpec((D, D), lambda i: (0, 0)),
          pl.BlockSpec((1, D), lambda i: (0, 0)),
      ],
      out_specs=pl.BlockSpec((_BLK, D), lambda i: (i, 0)),
      out_shape=jax.ShapeDtypeStruct((N, D), jnp.float32),
  )(x, W, b)


def _linear2(p0, p1, W, b):
  return pl.pallas_call(
      _lin2_body,
      grid=(N // _BLK,),
      in_specs=[
          pl.BlockSpec((_BLK, D), lambda i: (i, 0)),
          pl.BlockSpec((_BLK, D), lambda i: (i, 0)),
          pl.BlockSpec((D, D), lambda i: (0, 0)),
          pl.BlockSpec((1, D), lambda i: (0, 0)),
      ],
      out_specs=pl.BlockSpec((_BLK, D), lambda i: (i, 0)),
      out_shape=jax.ShapeDtypeStruct((N, D), jnp.float32),
  )(p0, p1, W, b)


def _add(a, b):
  return pl.pallas_call(
      _add_body,
      grid=(N // _BLK,),
      in_specs=[
          pl.BlockSpec((_BLK, D), lambda i: (i, 0)),
          pl.BlockSpec((_BLK, D), lambda i: (i, 0)),
      ],
      out_specs=pl.BlockSpec((_BLK, D), lambda i: (i, 0)),
      out_shape=jax.ShapeDtypeStruct((N, D), jnp.float32),
  )(a, b)


def kernel(x, edge_index, W1, b1, W2, b2):
  dst = edge_index[0]
  src = edge_index[1]
  pad = E_PAD - E
  src_p = jnp.concatenate([src, jnp.zeros((pad,), jnp.int32)])
  dst_p = jnp.concatenate([dst, jnp.full((pad,), N, jnp.int32)])
  b1r = b1.reshape(1, D)
  b2r = b2.reshape(1, D)

  h1 = _linear1(x, W1, b1r)
  P1 = _spmm_sc(h1, src_p, dst_p)
  h2 = _linear2(P1[0], P1[1], W2, b2r)
  P2 = _spmm_sc(h2, src_p, dst_p)
  return _add(P2[0], P2[1])
